# baseline (speedup 1.0000x reference)
import functools

import jax
import jax.numpy as jnp
from jax import lax
from jax.experimental import pallas as pl
from jax.experimental.pallas import tpu as pltpu
from jax.experimental.pallas import tpu_sc as plsc

N = 10000
E = 320000
D = 128
H = 128
C = 16
L = 4

NC = 2
NS = 16
NW = NC * NS
EPW = E // NW
CH = 40
NBUF = 5
CPB = 25
G = CPB // NBUF
NBLK = EPW // (CPB * CH)
NP = 10112
RPT = NP // NS


def _make_agg():
    mesh = plsc.VectorSubcoreMesh(core_axis_name="c", subcore_axis_name="s")

    @functools.partial(
        pl.kernel,
        out_type=jax.ShapeDtypeStruct((NC, NP, D), jnp.float32),
        mesh=mesh,
        scratch_types=[
            pltpu.VMEM((2, CPB, CH), jnp.int32),
            pltpu.VMEM((2, CPB, CH), jnp.int32),
            pltpu.VMEM((NBUF, CH, D), jnp.float32),
            pltpu.VMEM_SHARED((NP, D), jnp.float32),
        ] + [pltpu.SemaphoreType.DMA] * (2 * NBUF + 2),
    )
    def agg(h_hbm, ei_hbm, zero_hbm, out_hbm,
            sidx, didx, rows, acc, *sems):
        gsem = sems[:NBUF]
        ssem = sems[NBUF:2 * NBUF]
        isem = sems[2 * NBUF:]
        c = lax.axis_index("c")
        s = lax.axis_index("s")
        wid = s * NC + c
        row0 = s * RPT
        pltpu.async_copy(ei_hbm.at[0, wid, 0], sidx.at[0], isem[0])
        pltpu.async_copy(ei_hbm.at[1, wid, 0], didx.at[0], isem[0])
        pltpu.sync_copy(zero_hbm, rows.at[0])
        for k in range(RPT // CH):
            pltpu.sync_copy(rows.at[0], acc.at[pl.ds(row0 + k * CH, CH)])
        pltpu.sync_copy(rows.at[0, pl.ds(0, RPT % CH)],
                        acc.at[pl.ds(row0 + (RPT // CH) * CH, RPT % CH)])
        plsc.subcore_barrier()

        for b in range(NBLK):
            sl = b % 2
            pltpu.make_async_copy(ei_hbm.at[0, wid, b], sidx.at[sl],
                                  isem[sl]).wait()
            pltpu.make_async_copy(ei_hbm.at[1, wid, b], didx.at[sl],
                                  isem[sl]).wait()
            if b + 1 < NBLK:
                pltpu.async_copy(ei_hbm.at[0, wid, b + 1], sidx.at[1 - sl],
                                 isem[1 - sl])
                pltpu.async_copy(ei_hbm.at[1, wid, b + 1], didx.at[1 - sl],
                                 isem[1 - sl])
            for j in range(NBUF):
                pltpu.async_copy(h_hbm.at[sidx.at[sl, j]], rows.at[j],
                                 gsem[j])

            def group(g, carry):
                for j in range(NBUF):
                    i = g * NBUF + j
                    pltpu.make_async_copy(
                        zero_hbm, rows.at[j], gsem[j]).wait()
                    pltpu.async_copy(rows.at[j], acc.at[didx.at[sl, i]],
                                     ssem[j], add=True)

                @pl.when(g < G - 1)
                def _refill():
                    for j in range(NBUF):
                        i2 = (g + 1) * NBUF + j
                        pltpu.make_async_copy(
                            zero_hbm, rows.at[j], ssem[j]).wait()
                        pltpu.async_copy(h_hbm.at[sidx.at[sl, i2]],
                                         rows.at[j], gsem[j])
                return carry

            lax.fori_loop(0, G, group, 0)
            for j in range(NBUF):
                pltpu.make_async_copy(zero_hbm, rows.at[j], ssem[j]).wait()
        plsc.subcore_barrier()
        for k in range(4):
            pltpu.sync_copy(acc.at[pl.ds(row0 + k * 128, 128)],
                            out_hbm.at[c, pl.ds(row0 + k * 128, 128)])
        pltpu.sync_copy(acc.at[pl.ds(row0 + 512, 120)],
                        out_hbm.at[c, pl.ds(row0 + 512, 120)])

    return agg


_agg = _make_agg()


def _dot(a, b):
    return jnp.dot(a, b, preferred_element_type=jnp.float32,
                   precision=lax.Precision.HIGHEST)


def _bn(x):
    mean = jnp.mean(x, axis=0, keepdims=True)
    var = jnp.mean((x - mean) ** 2, axis=0, keepdims=True)
    return (x - mean) * lax.rsqrt(var + 1e-5)


def _embed_body(h_ref, w_ref, b_ref, out_ref, pooled_ref):
    h0 = _dot(h_ref[...], w_ref[...].T) + b_ref[...]
    out_ref[...] = h0
    pooled_ref[...] = jnp.sum(h0, axis=0, keepdims=True)


BR = 1000
NB = N // BR


def _layer_body(h_ref, parts_ref, sn_ref, w1_ref, b1_ref, w2_ref, b2_ref,
                eps_ref, hout_ref, pooled_ref, y1_ref, y2_ref, st_ref):
    p = pl.program_id(0)
    r = pl.program_id(1)
    rows = pl.ds(r * BR, BR)

    def _stats(k):
        m = st_ref[k:k + 1, :] * (1.0 / N)
        v = st_ref[k + 1:k + 2, :] * (1.0 / N) - m * m
        return m, lax.rsqrt(v + 1e-5)

    @pl.when(p == 0)
    def _p0():
        @pl.when(r == 0)
        def _init():
            st_ref[...] = jnp.zeros_like(st_ref)
        hh = ((1.0 + eps_ref[0, 0]) * h_ref[...]
              + parts_ref[0] + parts_ref[1])
        y1 = _dot(hh, w1_ref[...].T) + b1_ref[...]
        y1_ref[rows, :] = y1
        st_ref[0:1, :] += jnp.sum(y1, axis=0, keepdims=True)
        st_ref[1:2, :] += jnp.sum(y1 * y1, axis=0, keepdims=True)

    @pl.when(p == 1)
    def _p1():
        m, rs = _stats(0)
        z = jax.nn.relu((y1_ref[rows, :] - m) * rs)
        y2 = _dot(z, w2_ref[...].T) + b2_ref[...]
        y2_ref[rows, :] = y2
        st_ref[2:3, :] += jnp.sum(y2, axis=0, keepdims=True)
        st_ref[3:4, :] += jnp.sum(y2 * y2, axis=0, keepdims=True)

    @pl.when(p == 2)
    def _p2():
        m, rs = _stats(2)
        w = jax.nn.relu((y2_ref[rows, :] - m) * rs)
        u = w * sn_ref[...]
        y1_ref[rows, :] = u
        st_ref[4:5, :] += jnp.sum(u, axis=0, keepdims=True)
        st_ref[5:6, :] += jnp.sum(u * u, axis=0, keepdims=True)

    @pl.when(p == 3)
    def _p3():
        m, rs = _stats(4)
        ho = h_ref[...] + jax.nn.relu((y1_ref[rows, :] - m) * rs)
        hout_ref[...] = ho
        @pl.when(r == 0)
        def _initp():
            st_ref[6:7, :] = jnp.zeros_like(st_ref[6:7, :])
        st_ref[6:7, :] += jnp.sum(ho, axis=0, keepdims=True)
        @pl.when(r == NB - 1)
        def _fin():
            pooled_ref[...] = st_ref[6:7, :]


def _readout_body(pool_ref, wp_ref, bp_ref, out_ref):
    acc = jnp.zeros((1, C), dtype=jnp.float32)
    for i in range(L + 1):
        acc = acc + _dot(pool_ref[i:i + 1, :], wp_ref[i].T) + bp_ref[i:i + 1, :]
    out_ref[...] = acc


_embed = pl.pallas_call(
    _embed_body,
    out_shape=[jax.ShapeDtypeStruct((N, H), jnp.float32),
               jax.ShapeDtypeStruct((1, H), jnp.float32)],
)

_layer = pl.pallas_call(
    _layer_body,
    grid=(4, NB),
    in_specs=[
        pl.BlockSpec((BR, H), lambda p, r: (jnp.where((p == 0) | (p == 3), r, 0), 0)),
        pl.BlockSpec((2, BR, H), lambda p, r: (0, jnp.where(p == 0, r, 0), 0)),
        pl.BlockSpec((BR, 1), lambda p, r: (jnp.where(p == 2, r, 0), 0)),
        pl.BlockSpec((H, H), lambda p, r: (0, 0)),
        pl.BlockSpec((1, H), lambda p, r: (0, 0)),
        pl.BlockSpec((H, H), lambda p, r: (0, 0)),
        pl.BlockSpec((1, H), lambda p, r: (0, 0)),
        pl.BlockSpec((1, 1), lambda p, r: (0, 0)),
    ],
    out_specs=[
        pl.BlockSpec((BR, H), lambda p, r: (jnp.where(p == 3, r, 0), 0)),
        pl.BlockSpec((1, H), lambda p, r: (0, 0)),
    ],
    out_shape=[jax.ShapeDtypeStruct((N, H), jnp.float32),
               jax.ShapeDtypeStruct((1, H), jnp.float32)],
    scratch_shapes=[
        pltpu.VMEM((N, H), jnp.float32),
        pltpu.VMEM((N, H), jnp.float32),
        pltpu.VMEM((8, H), jnp.float32),
    ],
)

_readout = pl.pallas_call(
    _readout_body,
    out_shape=jax.ShapeDtypeStruct((1, C), jnp.float32),
)


def kernel(h, edge_index, e, snorm_n, snorm_e, W_emb, b_emb,
           W1, b1, W2, b2, eps, Wp, bp):
    ei5 = edge_index.reshape(2, NW, NBLK, CPB, CH)
    zeros = jnp.zeros((CH, D), dtype=jnp.float32)
    b_emb2 = b_emb.reshape(1, H)
    b1_2 = b1.reshape(L, 1, H)
    b2_2 = b2.reshape(L, 1, H)

    h0, pooled0 = _embed(h, W_emb, b_emb2)
    pooled = [pooled0]
    hcur = h0
    for i in range(L):
        parts = _agg(hcur, ei5, zeros)
        hcur, pi = _layer(hcur, parts, snorm_n, W1[i], b1_2[i],
                          W2[i], b2_2[i], eps[i].reshape(1, 1))
        pooled.append(pi)
    pool_all = jnp.concatenate(pooled, axis=0)
    return _readout(pool_all, Wp, bp)


# speedup vs baseline: 1.0463x; 1.0463x over previous
import functools

import jax
import jax.numpy as jnp
from jax import lax
from jax.experimental import pallas as pl
from jax.experimental.pallas import tpu as pltpu
from jax.experimental.pallas import tpu_sc as plsc

N = 10000
E = 320000
D = 128
H = 128
C = 16
L = 4

NC = 2
NS = 16
NW = NC * NS
EPW = E // NW
CH = 40
NBUF = 5
CPB = 25
G = CPB // NBUF
NBLK = EPW // (CPB * CH)
NP = 10112
RPT = NP // NS


def _make_agg():
    mesh = plsc.VectorSubcoreMesh(core_axis_name="c", subcore_axis_name="s")

    @functools.partial(
        pl.kernel,
        out_type=jax.ShapeDtypeStruct((NC, NP, D), jnp.float32),
        mesh=mesh,
        scratch_types=[
            pltpu.VMEM((2, CPB, CH), jnp.int32),
            pltpu.VMEM((2, CPB, CH), jnp.int32),
            pltpu.VMEM((NBUF, CH, D), jnp.float32),
            pltpu.VMEM_SHARED((NP, D), jnp.float32),
        ] + [pltpu.SemaphoreType.DMA] * (2 * NBUF + 3),
    )
    def agg(h_hbm, ei_hbm, zero_hbm, out_hbm,
            sidx, didx, rows, acc, *sems):
        gsem = sems[:NBUF]
        ssem = sems[NBUF:2 * NBUF]
        isem = sems[2 * NBUF:2 * NBUF + 2]
        zsem = sems[2 * NBUF + 2]
        c = lax.axis_index("c")
        s = lax.axis_index("s")
        wid = s * NC + c
        row0 = s * RPT
        pltpu.async_copy(ei_hbm.at[0, wid, 0], sidx.at[0], isem[0])
        pltpu.async_copy(ei_hbm.at[1, wid, 0], didx.at[0], isem[0])
        pltpu.sync_copy(zero_hbm, rows.at[0])
        for k in range(RPT // CH):
            pltpu.async_copy(rows.at[0], acc.at[pl.ds(row0 + k * CH, CH)],
                             zsem)
        pltpu.async_copy(rows.at[0, pl.ds(0, RPT % CH)],
                         acc.at[pl.ds(row0 + (RPT // CH) * CH, RPT % CH)],
                         zsem)
        for k in range(RPT // CH):
            pltpu.make_async_copy(rows.at[0],
                                  acc.at[pl.ds(row0 + k * CH, CH)],
                                  zsem).wait()
        pltpu.make_async_copy(rows.at[0, pl.ds(0, RPT % CH)],
                              acc.at[pl.ds(row0 + (RPT // CH) * CH,
                                           RPT % CH)], zsem).wait()
        plsc.subcore_barrier()

        for b in range(NBLK):
            sl = b % 2
            pltpu.make_async_copy(ei_hbm.at[0, wid, b], sidx.at[sl],
                                  isem[sl]).wait()
            pltpu.make_async_copy(ei_hbm.at[1, wid, b], didx.at[sl],
                                  isem[sl]).wait()
            if b + 1 < NBLK:
                pltpu.async_copy(ei_hbm.at[0, wid, b + 1], sidx.at[1 - sl],
                                 isem[1 - sl])
                pltpu.async_copy(ei_hbm.at[1, wid, b + 1], didx.at[1 - sl],
                                 isem[1 - sl])
            for j in range(NBUF):
                pltpu.async_copy(h_hbm.at[sidx.at[sl, j]], rows.at[j],
                                 gsem[j])

            def group(g, carry):
                for j in range(NBUF):
                    i = g * NBUF + j
                    pltpu.make_async_copy(
                        zero_hbm, rows.at[j], gsem[j]).wait()
                    pltpu.async_copy(rows.at[j], acc.at[didx.at[sl, i]],
                                     ssem[j], add=True)

                @pl.when(g < G - 1)
                def _refill():
                    for j in range(NBUF):
                        i2 = (g + 1) * NBUF + j
                        pltpu.make_async_copy(
                            zero_hbm, rows.at[j], ssem[j]).wait()
                        pltpu.async_copy(h_hbm.at[sidx.at[sl, i2]],
                                         rows.at[j], gsem[j])
                return carry

            lax.fori_loop(0, G, group, 0)
            for j in range(NBUF):
                pltpu.make_async_copy(zero_hbm, rows.at[j], ssem[j]).wait()
        plsc.subcore_barrier()
        for k in range(4):
            pltpu.async_copy(acc.at[pl.ds(row0 + k * 128, 128)],
                             out_hbm.at[c, pl.ds(row0 + k * 128, 128)], zsem)
        pltpu.async_copy(acc.at[pl.ds(row0 + 512, 120)],
                         out_hbm.at[c, pl.ds(row0 + 512, 120)], zsem)
        for k in range(4):
            pltpu.make_async_copy(acc.at[pl.ds(row0 + k * 128, 128)],
                                  out_hbm.at[c, pl.ds(row0 + k * 128, 128)],
                                  zsem).wait()
        pltpu.make_async_copy(acc.at[pl.ds(row0 + 512, 120)],
                              out_hbm.at[c, pl.ds(row0 + 512, 120)],
                              zsem).wait()

    return agg


_agg = _make_agg()


def _dot(a, b):
    return jnp.dot(a, b, preferred_element_type=jnp.float32,
                   precision=lax.Precision.HIGHEST)


def _bn(x):
    mean = jnp.mean(x, axis=0, keepdims=True)
    var = jnp.mean((x - mean) ** 2, axis=0, keepdims=True)
    return (x - mean) * lax.rsqrt(var + 1e-5)


def _embed_body(h_ref, w_ref, b_ref, out_ref, pooled_ref):
    h0 = _dot(h_ref[...], w_ref[...].T) + b_ref[...]
    out_ref[...] = h0
    pooled_ref[...] = jnp.sum(h0, axis=0, keepdims=True)


def _layer_body(h_ref, parts_ref, sn_ref, w1_ref, b1_ref, w2_ref, b2_ref,
                eps_ref, hout_ref, pooled_ref):
    h = h_ref[...]
    neigh = parts_ref[0, :N, :] + parts_ref[1, :N, :]
    hh = (1.0 + eps_ref[0, 0]) * h + neigh
    y = _dot(hh, w1_ref[...].T) + b1_ref[...]
    y = jax.nn.relu(_bn(y))
    y = _dot(y, w2_ref[...].T) + b2_ref[...]
    y = jax.nn.relu(_bn(y))
    y = y * sn_ref[...]
    y = jax.nn.relu(_bn(y))
    h_out = h + y
    hout_ref[...] = h_out
    pooled_ref[...] = jnp.sum(h_out, axis=0, keepdims=True)


def _readout_body(pool_ref, wp_ref, bp_ref, out_ref):
    acc = jnp.zeros((1, C), dtype=jnp.float32)
    for i in range(L + 1):
        acc = acc + _dot(pool_ref[i:i + 1, :], wp_ref[i].T) + bp_ref[i:i + 1, :]
    out_ref[...] = acc


_embed = pl.pallas_call(
    _embed_body,
    out_shape=[jax.ShapeDtypeStruct((N, H), jnp.float32),
               jax.ShapeDtypeStruct((1, H), jnp.float32)],
)

_layer = pl.pallas_call(
    _layer_body,
    out_shape=[jax.ShapeDtypeStruct((N, H), jnp.float32),
               jax.ShapeDtypeStruct((1, H), jnp.float32)],
)

_readout = pl.pallas_call(
    _readout_body,
    out_shape=jax.ShapeDtypeStruct((1, C), jnp.float32),
)


def kernel(h, edge_index, e, snorm_n, snorm_e, W_emb, b_emb,
           W1, b1, W2, b2, eps, Wp, bp):
    ei5 = edge_index.reshape(2, NW, NBLK, CPB, CH)
    zeros = jnp.zeros((CH, D), dtype=jnp.float32)
    b_emb2 = b_emb.reshape(1, H)
    b1_2 = b1.reshape(L, 1, H)
    b2_2 = b2.reshape(L, 1, H)

    h0, pooled0 = _embed(h, W_emb, b_emb2)
    pooled = [pooled0]
    hcur = h0
    for i in range(L):
        parts = _agg(hcur, ei5, zeros)
        hcur, pi = _layer(hcur, parts, snorm_n, W1[i], b1_2[i],
                          W2[i], b2_2[i], eps[i].reshape(1, 1))
        pooled.append(pi)
    pool_all = jnp.concatenate(pooled, axis=0)
    return _readout(pool_all, Wp, bp)


# speedup vs baseline: 1.1500x; 1.0992x over previous
import functools

import jax
import jax.numpy as jnp
from jax import lax
from jax.experimental import pallas as pl
from jax.experimental.pallas import tpu as pltpu
from jax.experimental.pallas import tpu_sc as plsc

N = 10000
E = 320000
D = 128
H = 128
C = 16
L = 4

NC = 2
NS = 16
NW = NC * NS
EPW = E // NW
CH = 40
NBUF = 5
CPB = 25
G = CPB // NBUF
NBLK = EPW // (CPB * CH)
NP = 10112
RPT = NP // NS


def _make_agg():
    mesh = plsc.VectorSubcoreMesh(core_axis_name="c", subcore_axis_name="s")

    @functools.partial(
        pl.kernel,
        out_type=jax.ShapeDtypeStruct((NC, NP, D), jnp.float32),
        mesh=mesh,
        scratch_types=[
            pltpu.VMEM((2, CPB, CH), jnp.int32),
            pltpu.VMEM((2, CPB, CH), jnp.int32),
            pltpu.VMEM((NBUF, CH, D), jnp.float32),
            pltpu.VMEM_SHARED((NP, D), jnp.float32),
        ] + [pltpu.SemaphoreType.DMA] * (2 * NBUF + 3),
    )
    def agg(h_hbm, ei_hbm, zero_hbm, out_hbm,
            sidx, didx, rows, acc, *sems):
        gsem = sems[:NBUF]
        ssem = sems[NBUF:2 * NBUF]
        isem = sems[2 * NBUF:2 * NBUF + 2]
        zsem = sems[2 * NBUF + 2]
        c = lax.axis_index("c")
        s = lax.axis_index("s")
        wid = s * NC + c
        row0 = s * RPT
        pltpu.async_copy(ei_hbm.at[0, wid, 0], sidx.at[0], isem[0])
        pltpu.async_copy(ei_hbm.at[1, wid, 0], didx.at[0], isem[0])
        pltpu.sync_copy(zero_hbm, rows.at[0])
        for k in range(RPT // CH):
            pltpu.async_copy(rows.at[0], acc.at[pl.ds(row0 + k * CH, CH)],
                             zsem)
        pltpu.async_copy(rows.at[0, pl.ds(0, RPT % CH)],
                         acc.at[pl.ds(row0 + (RPT // CH) * CH, RPT % CH)],
                         zsem)
        for k in range(RPT // CH):
            pltpu.make_async_copy(rows.at[0],
                                  acc.at[pl.ds(row0 + k * CH, CH)],
                                  zsem).wait()
        pltpu.make_async_copy(rows.at[0, pl.ds(0, RPT % CH)],
                              acc.at[pl.ds(row0 + (RPT // CH) * CH,
                                           RPT % CH)], zsem).wait()
        plsc.subcore_barrier()

        for b in range(NBLK):
            sl = b % 2
            pltpu.make_async_copy(ei_hbm.at[0, wid, b], sidx.at[sl],
                                  isem[sl]).wait()
            pltpu.make_async_copy(ei_hbm.at[1, wid, b], didx.at[sl],
                                  isem[sl]).wait()
            if b + 1 < NBLK:
                pltpu.async_copy(ei_hbm.at[0, wid, b + 1], sidx.at[1 - sl],
                                 isem[1 - sl])
                pltpu.async_copy(ei_hbm.at[1, wid, b + 1], didx.at[1 - sl],
                                 isem[1 - sl])
            for j in range(NBUF):
                pltpu.async_copy(h_hbm.at[sidx.at[sl, j]], rows.at[j],
                                 gsem[j])

            def group(g, carry):
                for j in range(NBUF):
                    i = g * NBUF + j
                    pltpu.make_async_copy(
                        zero_hbm, rows.at[j], gsem[j]).wait()
                    pltpu.async_copy(rows.at[j], acc.at[didx.at[sl, i]],
                                     ssem[j], add=True)

                @pl.when(g < G - 1)
                def _refill():
                    for j in range(NBUF):
                        i2 = (g + 1) * NBUF + j
                        pltpu.make_async_copy(
                            zero_hbm, rows.at[j], ssem[j]).wait()
                        pltpu.async_copy(h_hbm.at[sidx.at[sl, i2]],
                                         rows.at[j], gsem[j])
                return carry

            lax.fori_loop(0, G, group, 0)
            for j in range(NBUF):
                pltpu.make_async_copy(zero_hbm, rows.at[j], ssem[j]).wait()
        plsc.subcore_barrier()
        for k in range(4):
            pltpu.async_copy(acc.at[pl.ds(row0 + k * 128, 128)],
                             out_hbm.at[c, pl.ds(row0 + k * 128, 128)], zsem)
        pltpu.async_copy(acc.at[pl.ds(row0 + 512, 120)],
                         out_hbm.at[c, pl.ds(row0 + 512, 120)], zsem)
        for k in range(4):
            pltpu.make_async_copy(acc.at[pl.ds(row0 + k * 128, 128)],
                                  out_hbm.at[c, pl.ds(row0 + k * 128, 128)],
                                  zsem).wait()
        pltpu.make_async_copy(acc.at[pl.ds(row0 + 512, 120)],
                              out_hbm.at[c, pl.ds(row0 + 512, 120)],
                              zsem).wait()

    return agg


_agg = _make_agg()


def _dot(a, b):
    return jnp.dot(a, b, preferred_element_type=jnp.float32)


def _bn(x):
    mean = jnp.mean(x, axis=0, keepdims=True)
    var = jnp.mean((x - mean) ** 2, axis=0, keepdims=True)
    return (x - mean) * lax.rsqrt(var + 1e-5)


def _embed_body(h_ref, w_ref, b_ref, out_ref, pooled_ref):
    h0 = _dot(h_ref[...], w_ref[...].T) + b_ref[...]
    out_ref[...] = h0
    pooled_ref[...] = jnp.sum(h0, axis=0, keepdims=True)


def _layer_body(h_ref, parts_ref, sn_ref, w1_ref, b1_ref, w2_ref, b2_ref,
                eps_ref, hout_ref, pooled_ref):
    h = h_ref[...]
    neigh = parts_ref[0, :N, :] + parts_ref[1, :N, :]
    hh = (1.0 + eps_ref[0, 0]) * h + neigh
    y = _dot(hh, w1_ref[...].T) + b1_ref[...]
    y = jax.nn.relu(_bn(y))
    y = _dot(y, w2_ref[...].T) + b2_ref[...]
    y = jax.nn.relu(_bn(y))
    y = y * sn_ref[...]
    y = jax.nn.relu(_bn(y))
    h_out = h + y
    hout_ref[...] = h_out
    pooled_ref[...] = jnp.sum(h_out, axis=0, keepdims=True)


def _readout_body(pool_ref, wp_ref, bp_ref, out_ref):
    acc = jnp.zeros((1, C), dtype=jnp.float32)
    for i in range(L + 1):
        acc = acc + _dot(pool_ref[i:i + 1, :], wp_ref[i].T) + bp_ref[i:i + 1, :]
    out_ref[...] = acc


_embed = pl.pallas_call(
    _embed_body,
    out_shape=[jax.ShapeDtypeStruct((N, H), jnp.float32),
               jax.ShapeDtypeStruct((1, H), jnp.float32)],
)

_layer = pl.pallas_call(
    _layer_body,
    out_shape=[jax.ShapeDtypeStruct((N, H), jnp.float32),
               jax.ShapeDtypeStruct((1, H), jnp.float32)],
)

_readout = pl.pallas_call(
    _readout_body,
    out_shape=jax.ShapeDtypeStruct((1, C), jnp.float32),
)


def kernel(h, edge_index, e, snorm_n, snorm_e, W_emb, b_emb,
           W1, b1, W2, b2, eps, Wp, bp):
    ei5 = edge_index.reshape(2, NW, NBLK, CPB, CH)
    zeros = jnp.zeros((CH, D), dtype=jnp.float32)
    b_emb2 = b_emb.reshape(1, H)
    b1_2 = b1.reshape(L, 1, H)
    b2_2 = b2.reshape(L, 1, H)

    h0, pooled0 = _embed(h, W_emb, b_emb2)
    pooled = [pooled0]
    hcur = h0
    for i in range(L):
        parts = _agg(hcur, ei5, zeros)
        hcur, pi = _layer(hcur, parts, snorm_n, W1[i], b1_2[i],
                          W2[i], b2_2[i], eps[i].reshape(1, 1))
        pooled.append(pi)
    pool_all = jnp.concatenate(pooled, axis=0)
    return _readout(pool_all, Wp, bp)


# speedup vs baseline: 1.1842x; 1.0298x over previous
import functools

import jax
import jax.numpy as jnp
from jax import lax
from jax.experimental import pallas as pl
from jax.experimental.pallas import tpu as pltpu
from jax.experimental.pallas import tpu_sc as plsc

N = 10000
E = 320000
D = 128
H = 128
C = 16
L = 4

NC = 2
NS = 16
NW = NC * NS
EPW = E // NW
CH = 40
NBUF = 5
CPB = 25
G = CPB // NBUF
NBLK = EPW // (CPB * CH)
NP = 10112
RPT = NP // NS


def _make_agg():
    mesh = plsc.VectorSubcoreMesh(core_axis_name="c", subcore_axis_name="s")

    @functools.partial(
        pl.kernel,
        out_type=jax.ShapeDtypeStruct((NC, NP, D), jnp.float32),
        mesh=mesh,
        scratch_types=[
            pltpu.VMEM((2, CPB, CH), jnp.int32),
            pltpu.VMEM((2, CPB, CH), jnp.int32),
            pltpu.VMEM((NBUF, CH, D), jnp.float32),
            pltpu.VMEM_SHARED((NP, D), jnp.float32),
        ] + [pltpu.SemaphoreType.DMA] * (2 * NBUF + 3),
    )
    def agg(h_hbm, ei_hbm, zero_hbm, out_hbm,
            sidx, didx, rows, acc, *sems):
        gsem = sems[:NBUF]
        ssem = sems[NBUF:2 * NBUF]
        isem = sems[2 * NBUF:2 * NBUF + 2]
        zsem = sems[2 * NBUF + 2]
        c = lax.axis_index("c")
        s = lax.axis_index("s")
        wid = s * NC + c
        row0 = s * RPT

        def idx_fetch(b, sl, sem):
            pltpu.async_copy(ei_hbm.at[0, wid, b], sidx.at[sl], sem)
            pltpu.async_copy(ei_hbm.at[1, wid, b], didx.at[sl], sem)

        def idx_wait(b, sl, sem):
            pltpu.make_async_copy(ei_hbm.at[0, wid, b], sidx.at[sl],
                                  sem).wait()
            pltpu.make_async_copy(ei_hbm.at[1, wid, b], didx.at[sl],
                                  sem).wait()

        def gather(sl, i, j):
            pltpu.async_copy(h_hbm.at[sidx.at[sl, i]], rows.at[j], gsem[j])

        def gather_wait(j):
            pltpu.make_async_copy(zero_hbm, rows.at[j], gsem[j]).wait()

        def scatter(sl, i, j):
            pltpu.async_copy(rows.at[j], acc.at[didx.at[sl, i]], ssem[j],
                             add=True)

        def scatter_wait(j):
            pltpu.make_async_copy(zero_hbm, rows.at[j], ssem[j]).wait()

        idx_fetch(0, 0, isem[0])
        pltpu.sync_copy(zero_hbm, rows.at[0])
        for k in range(RPT // CH):
            pltpu.async_copy(rows.at[0], acc.at[pl.ds(row0 + k * CH, CH)],
                             zsem)
        pltpu.async_copy(rows.at[0, pl.ds(0, RPT % CH)],
                         acc.at[pl.ds(row0 + (RPT // CH) * CH, RPT % CH)],
                         zsem)
        for k in range(RPT // CH):
            pltpu.make_async_copy(rows.at[0],
                                  acc.at[pl.ds(row0 + k * CH, CH)],
                                  zsem).wait()
        pltpu.make_async_copy(rows.at[0, pl.ds(0, RPT % CH)],
                              acc.at[pl.ds(row0 + (RPT // CH) * CH,
                                           RPT % CH)], zsem).wait()
        plsc.subcore_barrier()

        idx_wait(0, 0, isem[0])
        idx_fetch(1, 1, isem[1])
        for j in range(NBUF):
            gather(0, j, j)

        for b in range(NBLK):
            sl = b % 2

            def group(g, carry):
                for j in range(NBUF):
                    gather_wait(j)
                    scatter(sl, g * NBUF + j, j)
                for j in range(NBUF):
                    scatter_wait(j)
                    gather(sl, (g + 1) * NBUF + j, j)
                return carry

            lax.fori_loop(0, G - 1, group, 0)
            for j in range(NBUF):
                gather_wait(j)
                scatter(sl, (G - 1) * NBUF + j, j)
            if b + 1 < NBLK:
                idx_wait(b + 1, 1 - sl, isem[1 - sl])
                for j in range(NBUF):
                    scatter_wait(j)
                    gather(1 - sl, j, j)
                if b + 2 < NBLK:
                    idx_fetch(b + 2, sl, isem[sl])
            else:
                for j in range(NBUF):
                    scatter_wait(j)
        plsc.subcore_barrier()
        for k in range(4):
            pltpu.async_copy(acc.at[pl.ds(row0 + k * 128, 128)],
                             out_hbm.at[c, pl.ds(row0 + k * 128, 128)], zsem)
        pltpu.async_copy(acc.at[pl.ds(row0 + 512, 120)],
                         out_hbm.at[c, pl.ds(row0 + 512, 120)], zsem)
        for k in range(4):
            pltpu.make_async_copy(acc.at[pl.ds(row0 + k * 128, 128)],
                                  out_hbm.at[c, pl.ds(row0 + k * 128, 128)],
                                  zsem).wait()
        pltpu.make_async_copy(acc.at[pl.ds(row0 + 512, 120)],
                              out_hbm.at[c, pl.ds(row0 + 512, 120)],
                              zsem).wait()

    return agg


_agg = _make_agg()


def _dot(a, b):
    return jnp.dot(a, b, preferred_element_type=jnp.float32)


def _bn(x):
    mean = jnp.mean(x, axis=0, keepdims=True)
    var = jnp.mean((x - mean) ** 2, axis=0, keepdims=True)
    return (x - mean) * lax.rsqrt(var + 1e-5)


def _embed_body(h_ref, w_ref, b_ref, out_ref, pooled_ref):
    h0 = _dot(h_ref[...], w_ref[...].T) + b_ref[...]
    out_ref[...] = h0
    pooled_ref[...] = jnp.sum(h0, axis=0, keepdims=True)


def _layer_body(h_ref, parts_ref, sn_ref, w1_ref, b1_ref, w2_ref, b2_ref,
                eps_ref, hout_ref, pooled_ref):
    h = h_ref[...]
    neigh = parts_ref[0, :N, :] + parts_ref[1, :N, :]
    hh = (1.0 + eps_ref[0, 0]) * h + neigh
    y = _dot(hh, w1_ref[...].T) + b1_ref[...]
    y = jax.nn.relu(_bn(y))
    y = _dot(y, w2_ref[...].T) + b2_ref[...]
    y = jax.nn.relu(_bn(y))
    y = y * sn_ref[...]
    y = jax.nn.relu(_bn(y))
    h_out = h + y
    hout_ref[...] = h_out
    pooled_ref[...] = jnp.sum(h_out, axis=0, keepdims=True)


def _readout_body(pool_ref, wp_ref, bp_ref, out_ref):
    acc = jnp.zeros((1, C), dtype=jnp.float32)
    for i in range(L + 1):
        acc = acc + _dot(pool_ref[i:i + 1, :], wp_ref[i].T) + bp_ref[i:i + 1, :]
    out_ref[...] = acc


_embed = pl.pallas_call(
    _embed_body,
    out_shape=[jax.ShapeDtypeStruct((N, H), jnp.float32),
               jax.ShapeDtypeStruct((1, H), jnp.float32)],
)

_layer = pl.pallas_call(
    _layer_body,
    out_shape=[jax.ShapeDtypeStruct((N, H), jnp.float32),
               jax.ShapeDtypeStruct((1, H), jnp.float32)],
)

_readout = pl.pallas_call(
    _readout_body,
    out_shape=jax.ShapeDtypeStruct((1, C), jnp.float32),
)


def kernel(h, edge_index, e, snorm_n, snorm_e, W_emb, b_emb,
           W1, b1, W2, b2, eps, Wp, bp):
    ei5 = edge_index.reshape(2, NW, NBLK, CPB, CH)
    zeros = jnp.zeros((CH, D), dtype=jnp.float32)
    b_emb2 = b_emb.reshape(1, H)
    b1_2 = b1.reshape(L, 1, H)
    b2_2 = b2.reshape(L, 1, H)

    h0, pooled0 = _embed(h, W_emb, b_emb2)
    pooled = [pooled0]
    hcur = h0
    for i in range(L):
        parts = _agg(hcur, ei5, zeros)
        hcur, pi = _layer(hcur, parts, snorm_n, W1[i], b1_2[i],
                          W2[i], b2_2[i], eps[i].reshape(1, 1))
        pooled.append(pi)
    pool_all = jnp.concatenate(pooled, axis=0)
    return _readout(pool_all, Wp, bp)
